# Initial kernel scaffold; baseline (speedup 1.0000x reference)
import jax
import jax.numpy as jnp
from jax.experimental import pallas as pl


def kernel(sender_input, W_dir, b_dir, W_dist, b_dist):
    raise NotImplementedError("write your pallas kernel here")


# speedup vs baseline: 1.1707x; 1.1707x over previous
import functools

import jax
import jax.numpy as jnp
from jax.experimental import pallas as pl
from jax.experimental.pallas import tpu as pltpu

B = 128
D = 128
V = 100000
TV = 2048
NT = (V + TV - 1) // TV

NEG = -1e30


def _body(x_ref, w_ref, b_ref, wd_ref, bd_ref,
          samp_ref, dist_ref, logp_ref, ent_ref,
          m_ref, s_ref, t_ref, idx_ref):
    g = pl.program_id(0)

    @pl.when(g == 0)
    def _init():
        m_ref[...] = jnp.full((B, 1), NEG, jnp.float32)
        s_ref[...] = jnp.zeros((B, 1), jnp.float32)
        t_ref[...] = jnp.zeros((B, 1), jnp.float32)
        idx_ref[...] = jnp.zeros((B, 1), jnp.int32)
        dist_ref[...] = (
            jnp.sum(x_ref[...] * wd_ref[...], axis=1, keepdims=True)
            + bd_ref[0, 0]
        )

    logits = (
        jnp.dot(x_ref[...], w_ref[...], preferred_element_type=jnp.float32)
        + b_ref[...]
    )
    cols = jax.lax.broadcasted_iota(jnp.int32, (B, TV), 1) + g * TV
    logits = jnp.where(cols < V, logits, NEG)

    tmax = jnp.max(logits, axis=1, keepdims=True)
    targ = jnp.min(
        jnp.where(logits == tmax, cols, jnp.int32(2**31 - 1)),
        axis=1, keepdims=True,
    )

    m_old = m_ref[...]
    m_new = jnp.maximum(m_old, tmax)
    alpha = jnp.exp(m_old - m_new)
    p = jnp.exp(logits - m_new)
    s_ref[...] = s_ref[...] * alpha + jnp.sum(p, axis=1, keepdims=True)
    t_ref[...] = t_ref[...] * alpha + jnp.sum(p * logits, axis=1, keepdims=True)
    idx_ref[...] = jnp.where(tmax > m_old, targ, idx_ref[...])
    m_ref[...] = m_new

    @pl.when(g == NT - 1)
    def _finish():
        m = m_ref[...]
        s = s_ref[...]
        logs = jnp.log(s)
        samp_ref[...] = idx_ref[...].astype(jnp.float32)
        logp_ref[...] = -logs
        ent_ref[...] = (m + logs) - t_ref[...] / s


@jax.jit
def kernel(sender_input, W_dir, b_dir, W_dist, b_dist):
    b2 = b_dir.reshape(1, V)
    wd_row = W_dist.reshape(1, D)
    bd2 = b_dist.reshape(1, 1)

    out = pl.pallas_call(
        _body,
        grid=(NT,),
        in_specs=[
            pl.BlockSpec((B, D), lambda g: (0, 0)),
            pl.BlockSpec((D, TV), lambda g: (0, g)),
            pl.BlockSpec((1, TV), lambda g: (0, g)),
            pl.BlockSpec((1, D), lambda g: (0, 0)),
            pl.BlockSpec((1, 1), lambda g: (0, 0)),
        ],
        out_specs=[
            pl.BlockSpec((B, 1), lambda g: (0, 0)),
            pl.BlockSpec((B, 1), lambda g: (0, 0)),
            pl.BlockSpec((B, 1), lambda g: (0, 0)),
            pl.BlockSpec((B, 1), lambda g: (0, 0)),
        ],
        out_shape=[
            jax.ShapeDtypeStruct((B, 1), jnp.float32),
            jax.ShapeDtypeStruct((B, 1), jnp.float32),
            jax.ShapeDtypeStruct((B, 1), jnp.float32),
            jax.ShapeDtypeStruct((B, 1), jnp.float32),
        ],
        scratch_shapes=[
            pltpu.VMEM((B, 1), jnp.float32),
            pltpu.VMEM((B, 1), jnp.float32),
            pltpu.VMEM((B, 1), jnp.float32),
            pltpu.VMEM((B, 1), jnp.int32),
        ],
        compiler_params=pltpu.CompilerParams(
            dimension_semantics=("arbitrary",),
        ),
    )(sender_input, W_dir, b2, wd_row, bd2)

    samp, dist, logp, ent = out
    message = jnp.concatenate([samp, dist], axis=1)
    return (message, logp[:, 0], ent[:, 0])


# speedup vs baseline: 1.3955x; 1.1921x over previous
import functools

import jax
import jax.numpy as jnp
from jax.experimental import pallas as pl
from jax.experimental.pallas import tpu as pltpu

B = 128
D = 128
V = 100000
TV = 8192
NT = (V + TV - 1) // TV

NEG = -1e30


def _body(x_ref, w_ref, b_ref, wd_ref, bd_ref,
          samp_ref, dist_ref, logp_ref, ent_ref,
          m_ref, s_ref, t_ref, idx_ref):
    g = pl.program_id(0)

    @pl.when(g == 0)
    def _init():
        m_ref[...] = jnp.full((B, 1), NEG, jnp.float32)
        s_ref[...] = jnp.zeros((B, 1), jnp.float32)
        t_ref[...] = jnp.zeros((B, 1), jnp.float32)
        idx_ref[...] = jnp.zeros((B, 1), jnp.int32)
        dist_ref[...] = (
            jnp.sum(x_ref[...] * wd_ref[...], axis=1, keepdims=True)
            + bd_ref[0, 0]
        )

    logits = (
        jnp.dot(x_ref[...], w_ref[...], preferred_element_type=jnp.float32)
        + b_ref[...]
    )
    cols = jax.lax.broadcasted_iota(jnp.int32, (B, TV), 1) + g * TV
    logits = jnp.where(cols < V, logits, NEG)

    tmax = jnp.max(logits, axis=1, keepdims=True)
    targ = jnp.min(
        jnp.where(logits == tmax, cols, jnp.int32(2**31 - 1)),
        axis=1, keepdims=True,
    )

    m_old = m_ref[...]
    m_new = jnp.maximum(m_old, tmax)
    alpha = jnp.exp(m_old - m_new)
    p = jnp.exp(logits - m_new)
    s_ref[...] = s_ref[...] * alpha + jnp.sum(p, axis=1, keepdims=True)
    t_ref[...] = t_ref[...] * alpha + jnp.sum(p * logits, axis=1, keepdims=True)
    idx_ref[...] = jnp.where(tmax > m_old, targ, idx_ref[...])
    m_ref[...] = m_new

    @pl.when(g == NT - 1)
    def _finish():
        m = m_ref[...]
        s = s_ref[...]
        logs = jnp.log(s)
        samp_ref[...] = idx_ref[...].astype(jnp.float32)
        logp_ref[...] = -logs
        ent_ref[...] = (m + logs) - t_ref[...] / s


@jax.jit
def kernel(sender_input, W_dir, b_dir, W_dist, b_dist):
    b2 = b_dir.reshape(1, V)
    wd_row = W_dist.reshape(1, D)
    bd2 = b_dist.reshape(1, 1)

    out = pl.pallas_call(
        _body,
        grid=(NT,),
        in_specs=[
            pl.BlockSpec((B, D), lambda g: (0, 0)),
            pl.BlockSpec((D, TV), lambda g: (0, g)),
            pl.BlockSpec((1, TV), lambda g: (0, g)),
            pl.BlockSpec((1, D), lambda g: (0, 0)),
            pl.BlockSpec((1, 1), lambda g: (0, 0)),
        ],
        out_specs=[
            pl.BlockSpec((B, 1), lambda g: (0, 0)),
            pl.BlockSpec((B, 1), lambda g: (0, 0)),
            pl.BlockSpec((B, 1), lambda g: (0, 0)),
            pl.BlockSpec((B, 1), lambda g: (0, 0)),
        ],
        out_shape=[
            jax.ShapeDtypeStruct((B, 1), jnp.float32),
            jax.ShapeDtypeStruct((B, 1), jnp.float32),
            jax.ShapeDtypeStruct((B, 1), jnp.float32),
            jax.ShapeDtypeStruct((B, 1), jnp.float32),
        ],
        scratch_shapes=[
            pltpu.VMEM((B, 1), jnp.float32),
            pltpu.VMEM((B, 1), jnp.float32),
            pltpu.VMEM((B, 1), jnp.float32),
            pltpu.VMEM((B, 1), jnp.int32),
        ],
        compiler_params=pltpu.CompilerParams(
            dimension_semantics=("arbitrary",),
        ),
    )(sender_input, W_dir, b2, wd_row, bd2)

    samp, dist, logp, ent = out
    message = jnp.concatenate([samp, dist], axis=1)
    return (message, logp[:, 0], ent[:, 0])
